# Initial kernel scaffold; baseline (speedup 1.0000x reference)
import jax
import jax.numpy as jnp
from jax.experimental import pallas as pl


def kernel(inputs, Wsel, bsel, W, b):
    raise NotImplementedError("write your pallas kernel here")


# speedup vs baseline: 1.1470x; 1.1470x over previous
import jax
import jax.numpy as jnp
from jax.experimental import pallas as pl

N_EXPERTS = 8
K = 2
N_OUT = 1024


def kernel(inputs, Wsel, bsel, W, b):
    xb = inputs.astype(jnp.bfloat16)
    gate_odds = jax.nn.softmax(
        jnp.dot(xb, Wsel.astype(jnp.bfloat16),
                preferred_element_type=jnp.float32) + bsel, axis=-1)
    row_sums = jnp.sum(gate_odds, axis=-1)
    aux_loss = jnp.var(row_sums) / (jnp.mean(row_sums) ** 2 + 1e-10)
    values, indices = jax.lax.top_k(gate_odds, K)
    out = jnp.zeros((inputs.shape[0], N_OUT), dtype=inputs.dtype)
    for i in range(N_EXPERTS):
        y_i = jax.nn.relu(
            jnp.dot(xb, W[i].astype(jnp.bfloat16),
                    preferred_element_type=jnp.float32) + b[i])
        gate_i = jnp.sum(values * (indices == i).astype(inputs.dtype), axis=-1)
        out = out + gate_i[:, None] * y_i
    return (out, aux_loss)


# speedup vs baseline: 1.6160x; 1.4088x over previous
import functools

import jax
import jax.numpy as jnp
from jax.experimental import pallas as pl
from jax.experimental.pallas import tpu as pltpu

N_TOKENS = 4096
N_IN = 1024
N_OUT = 1024
N_EXPERTS = 8
ROW_CHUNK = 1024


def _routing_body(x_ref, wsel_ref, bsel_ref, xb_ref, g_ref, aux_ref):
    xb = x_ref[...].astype(jnp.bfloat16)
    xb_ref[...] = xb
    wselb = wsel_ref[...].astype(jnp.bfloat16)
    logits = (
        jnp.dot(xb, wselb, preferred_element_type=jnp.float32) + bsel_ref[...]
    )
    m = jnp.max(logits, axis=-1, keepdims=True)
    e = jnp.exp(logits - m)
    s = jnp.sum(e, axis=-1, keepdims=True)
    p = e / s

    row_sums = jnp.sum(p, axis=-1)
    mean = jnp.mean(row_sums)
    var = jnp.mean((row_sums - mean) ** 2)
    aux_ref[...] = (var / (mean * mean + 1e-10)).reshape(1, 1)

    iota = jax.lax.broadcasted_iota(jnp.int32, p.shape, 1)
    max1 = jnp.max(p, axis=-1, keepdims=True)
    i1 = jnp.min(jnp.where(p == max1, iota, N_EXPERTS), axis=-1, keepdims=True)
    m1 = iota == i1
    p2 = jnp.where(m1, -1.0, p)
    max2 = jnp.max(p2, axis=-1, keepdims=True)
    i2 = jnp.min(jnp.where(p2 == max2, iota, N_EXPERTS), axis=-1, keepdims=True)
    m2 = iota == i2
    g_ref[...] = max1 * m1.astype(jnp.float32) + max2 * m2.astype(jnp.float32)


def _expert_body(xb_ref, w_ref, b_ref, g_ref, out_ref):
    i = pl.program_id(0)
    wb = w_ref[0].astype(jnp.bfloat16)
    iota = jax.lax.broadcasted_iota(jnp.int32, (N_TOKENS, N_EXPERTS), 1)
    g = jnp.sum(
        g_ref[...] * (iota == i).astype(jnp.float32), axis=-1, keepdims=True
    )
    for c in range(N_TOKENS // ROW_CHUNK):
        rows = pl.ds(c * ROW_CHUNK, ROW_CHUNK)
        y = (
            jnp.dot(xb_ref[rows, :], wb, preferred_element_type=jnp.float32)
            + b_ref[0]
        )
        gy = jnp.maximum(y, 0.0) * g[c * ROW_CHUNK:(c + 1) * ROW_CHUNK, :]

        @pl.when(i == 0)
        def _():
            out_ref[rows, :] = gy

        @pl.when(i != 0)
        def _():
            out_ref[rows, :] += gy


@functools.partial(jax.jit, static_argnames=())
def kernel(inputs, Wsel, bsel, W, b):
    xb, g, aux = pl.pallas_call(
        _routing_body,
        out_shape=(
            jax.ShapeDtypeStruct((N_TOKENS, N_IN), jnp.bfloat16),
            jax.ShapeDtypeStruct((N_TOKENS, N_EXPERTS), jnp.float32),
            jax.ShapeDtypeStruct((1, 1), jnp.float32),
        ),
    )(inputs, Wsel, bsel.reshape(1, N_EXPERTS))

    out = pl.pallas_call(
        _expert_body,
        grid=(N_EXPERTS,),
        in_specs=[
            pl.BlockSpec((N_TOKENS, N_IN), lambda i: (0, 0)),
            pl.BlockSpec((1, N_IN, N_OUT), lambda i: (i, 0, 0)),
            pl.BlockSpec((1, 1, N_OUT), lambda i: (i, 0, 0)),
            pl.BlockSpec((N_TOKENS, N_EXPERTS), lambda i: (0, 0)),
        ],
        out_specs=pl.BlockSpec((N_TOKENS, N_OUT), lambda i: (0, 0)),
        out_shape=jax.ShapeDtypeStruct((N_TOKENS, N_OUT), jnp.float32),
        compiler_params=pltpu.CompilerParams(
            dimension_semantics=("arbitrary",),
        ),
    )(xb, W, b.reshape(N_EXPERTS, 1, N_OUT), g)
    return (out, aux.reshape(()))
